# Initial kernel scaffold; baseline (speedup 1.0000x reference)
import jax
import jax.numpy as jnp
from jax.experimental import pallas as pl


def kernel(x, edge_index, W1, W2):
    raise NotImplementedError("write your pallas kernel here")


# speedup vs baseline: 55.0190x; 55.0190x over previous
import functools

import numpy as np

import jax
import jax.numpy as jnp
from jax import lax
from jax.experimental import pallas as pl
from jax.experimental.pallas import tpu as pltpu
from jax.experimental.pallas import tpu_sc as plsc

_N = 10000
_E = 320000
_C = 128
_K = 128
_NC = 2
_NS = 16
_NW = _NC * _NS
_EPT = _E // _NW
_CPT = 2 * (-(-_EPT // (2 * _K)))
_EPTP = _CPT * _K
_RPT = 624
_TAIL = _N - _RPT * _NS

_mesh = plsc.VectorSubcoreMesh(core_axis_name="c", subcore_axis_name="s")


@functools.partial(
    pl.kernel,
    out_type=jax.ShapeDtypeStruct((_NC, _N, _C), jnp.float32),
    mesh=_mesh,
    scratch_types=[
        pltpu.VMEM((2, _K), jnp.int32),
        pltpu.VMEM((2, _K), jnp.int32),
        pltpu.VMEM((_K, _C), jnp.float32),
        pltpu.VMEM((_K, _C), jnp.float32),
        pltpu.VMEM_SHARED((_N + 8, _C), jnp.float32),
        pltpu.SemaphoreType.DMA,
        pltpu.SemaphoreType.DMA,
        pltpu.SemaphoreType.DMA,
        pltpu.SemaphoreType.DMA,
    ],
)
def _aggregate(h_hbm, sd_hbm, out_hbm, idx_a, idx_b, rows_a, rows_b,
               acc, gsem_a, gsem_b, isem_a, isem_b):
    cid = lax.axis_index("c")
    sid = lax.axis_index("s")
    wid = cid * jnp.int32(_NS) + sid
    row0 = sid * jnp.int32(_RPT)

    zero = jnp.int32(0)
    one = jnp.int32(1)
    cmax = jnp.int32(_CPT - 1)

    pltpu.async_copy(sd_hbm.at[wid, zero], idx_a, isem_a)
    pltpu.async_copy(sd_hbm.at[wid, one], idx_b, isem_b)

    pltpu.sync_copy(h_hbm.at[pl.ds(row0, _RPT)],
                    acc.at[pl.ds(row0, _RPT)])

    @pl.when(sid == 0)
    def _():
        pltpu.sync_copy(h_hbm.at[pl.ds(_RPT * _NS, _TAIL)],
                        acc.at[pl.ds(_RPT * _NS, _TAIL)])

    plsc.subcore_barrier()

    def idx_wait(buf, sem):
        pltpu.make_async_copy(sd_hbm.at[wid, zero], buf, sem).wait()

    def body(i, _):
        c0 = i * jnp.int32(2)
        n0 = jnp.minimum(c0 + jnp.int32(2), cmax)
        n1 = jnp.minimum(c0 + jnp.int32(3), cmax)
        idx_wait(idx_a, isem_a)
        ga = pltpu.async_copy(h_hbm.at[idx_a.at[zero]], rows_a, gsem_a)
        idx_wait(idx_b, isem_b)
        gb = pltpu.async_copy(h_hbm.at[idx_b.at[zero]], rows_b, gsem_b)
        ga.wait()
        pltpu.sync_copy(rows_a, acc.at[idx_a.at[one]], add=True)
        pltpu.async_copy(sd_hbm.at[wid, n0], idx_a, isem_a)
        gb.wait()
        pltpu.sync_copy(rows_b, acc.at[idx_b.at[one]], add=True)
        pltpu.async_copy(sd_hbm.at[wid, n1], idx_b, isem_b)
        return i + one, None

    lax.scan(body, jnp.int32(0), None, length=_CPT // 2)

    idx_wait(idx_a, isem_a)
    idx_wait(idx_b, isem_b)

    plsc.subcore_barrier()

    pltpu.sync_copy(acc.at[pl.ds(row0, _RPT)],
                    out_hbm.at[cid, pl.ds(row0, _RPT)])

    @pl.when(sid == 0)
    def _():
        pltpu.sync_copy(acc.at[pl.ds(_RPT * _NS, _TAIL)],
                        out_hbm.at[cid, pl.ds(_RPT * _NS, _TAIL)])


_BLK = 400


def _zero():
    return jnp.int32(0)


def _mm_body(p_ref, h_ref, w_ref, o_ref):
    a = p_ref[0] + p_ref[1] - h_ref[...]
    o_ref[...] = jnp.maximum(
        lax.dot_general(a, w_ref[...], (((1,), (1,)), ((), ())),
                        preferred_element_type=jnp.float32,
                        precision=lax.Precision.HIGHEST),
        0.0)


def _mm(parts, h, w):
    return pl.pallas_call(
        _mm_body,
        grid=(_N // _BLK,),
        in_specs=[
            pl.BlockSpec((_NC, _BLK, _C), lambda i: (_zero(), i, _zero())),
            pl.BlockSpec((_BLK, _C), lambda i: (i, _zero())),
            pl.BlockSpec((_C, _C), lambda i: (_zero(), _zero())),
        ],
        out_specs=pl.BlockSpec((_BLK, _C), lambda i: (i, _zero())),
        out_shape=jax.ShapeDtypeStruct((_N, _C), jnp.float32),
    )(parts, h, w)


def kernel(x, edge_index, W1, W2):
    x = x.astype(jnp.float32)
    pad = ((0, 0), (0, _EPTP - _EPT))
    src = jnp.pad(edge_index[0].astype(jnp.int32).reshape(_NW, _EPT), pad,
                  constant_values=0).reshape(_NW, _CPT, _K)
    dst = jnp.pad(edge_index[1].astype(jnp.int32).reshape(_NW, _EPT), pad,
                  constant_values=_N).reshape(_NW, _CPT, _K)
    sd = jnp.stack([src, dst], axis=2)
    w1 = W1.astype(jnp.float32)
    w2 = W2.astype(jnp.float32)
    p1 = _aggregate(x, sd)
    h1 = _mm(p1, x, w1)
    p2 = _aggregate(h1, sd)
    h2 = _mm(p2, h1, w2)
    return h2.astype(jnp.float64)
